# baseline (speedup 1.0000x reference)
import functools

import jax
import jax.numpy as jnp
from jax import lax
from jax.experimental import pallas as pl
from jax.experimental.pallas import tpu as pltpu
from jax.experimental.pallas import tpu_sc as plsc

_BATCH = 16384
_EMB = 16
_NC = 2
_NS = 16
_NW = _NC * _NS
_BPW = _BATCH // _NW
_EPS = 1e-5

_mesh = plsc.VectorSubcoreMesh(core_axis_name="c", subcore_axis_name="s")


@functools.partial(
    pl.kernel,
    out_type=jax.ShapeDtypeStruct((_BATCH, _EMB), jnp.float32),
    mesh=_mesh,
    scratch_types=[
        pltpu.VMEM((_BPW,), jnp.int32),
        pltpu.VMEM((_BPW, _EMB), jnp.float32),
        pltpu.SemaphoreType.DMA,
    ],
    compiler_params=pltpu.CompilerParams(use_tc_tiling_on_sc=False),
)
def _sc_gather(idx_hbm, table_hbm, out_hbm, idx_v, rows_v, sem):
    wid = lax.axis_index("s") * _NC + lax.axis_index("c")
    base = wid * _BPW
    pltpu.sync_copy(idx_hbm.at[pl.ds(base, _BPW)], idx_v)
    pltpu.async_copy(table_hbm.at[idx_v], rows_v, sem).wait()
    pltpu.sync_copy(rows_v, out_hbm.at[pl.ds(base, _BPW)])


def _tc_body(uf_ref, ages_ref, w16t_ref, wage_ref, bias_ref, gb_ref, out_ref):
    a = ages_ref[...]
    n = jnp.float32(_BATCH)
    mean = jnp.sum(a) / n
    d = a - mean
    var = jnp.sum(d * d) / n
    gamma = gb_ref[0, 0]
    beta = gb_ref[0, 1]
    an = d * lax.rsqrt(var + _EPS) * gamma + beta
    an = jnp.maximum(an, 0.0)
    uf = jnp.maximum(uf_ref[...], 0.0)
    out_ref[...] = (
        jnp.dot(uf, w16t_ref[...], preferred_element_type=jnp.float32)
        + an * wage_ref[...]
        + bias_ref[...]
    )


def kernel(user_ids, ages, emb_table, bn_gamma, bn_beta, W, b):
    out_dim = W.shape[0]
    w_pad = jnp.zeros((_EMB, W.shape[1]), W.dtype).at[:out_dim].set(W)
    w16t = w_pad[:, :_EMB].T
    wage = w_pad[:, _EMB].reshape(1, _EMB)
    bias = jnp.zeros((1, _EMB), b.dtype).at[0, :out_dim].set(b)
    gb = jnp.stack([bn_gamma[0], bn_beta[0]]).reshape(1, 2)

    uf = _sc_gather(user_ids, emb_table)

    out_pad = pl.pallas_call(
        _tc_body,
        out_shape=jax.ShapeDtypeStruct((_BATCH, _EMB), jnp.float32),
    )(uf, ages.reshape(_BATCH, 1), w16t, wage, bias, gb)
    return out_pad[:, :out_dim]


# speedup vs baseline: 1.6249x; 1.6249x over previous
import functools

import jax
import jax.numpy as jnp
from jax import lax
from jax.experimental import pallas as pl
from jax.experimental.pallas import tpu as pltpu
from jax.experimental.pallas import tpu_sc as plsc

_BATCH = 16384
_EMB = 16
_NC = 2
_NS = 16
_NW = _NC * _NS
_BPW = _BATCH // _NW
_EPS = 1e-5

_mesh = plsc.VectorSubcoreMesh(core_axis_name="c", subcore_axis_name="s")


@functools.partial(
    pl.kernel,
    out_type=jax.ShapeDtypeStruct((_BATCH, _EMB), jnp.float32),
    mesh=_mesh,
    scratch_types=[
        pltpu.VMEM((_BPW,), jnp.int32),
        pltpu.VMEM((_BPW, _EMB), jnp.float32),
        pltpu.SemaphoreType.DMA,
    ],
)
def _sc_gather(idx_hbm, table_hbm, out_hbm, idx_v, rows_v, sem):
    wid = lax.axis_index("s") * _NC + lax.axis_index("c")
    base = wid * _BPW
    pltpu.sync_copy(idx_hbm.at[pl.ds(base, _BPW)], idx_v)

    def _fire(i, _):
        k0 = i * 16
        vec = idx_v[pl.ds(k0, 16)]
        for d in range(16):
            pltpu.make_async_copy(
                table_hbm.at[pl.ds(vec[d], 1), :],
                rows_v.at[pl.ds(k0 + d, 1), :],
                sem,
            ).start()
        return _

    lax.fori_loop(0, _BPW // 16, _fire, 0)

    def _drain(i, _):
        k0 = i * 16
        vec = idx_v[pl.ds(k0, 16)]
        for d in range(16):
            pltpu.make_async_copy(
                table_hbm.at[pl.ds(vec[d], 1), :],
                rows_v.at[pl.ds(k0 + d, 1), :],
                sem,
            ).wait()
        return _

    lax.fori_loop(0, _BPW // 16, _drain, 0)
    pltpu.sync_copy(rows_v, out_hbm.at[pl.ds(base, _BPW)])


def _tc_body(uf_ref, ages_ref, w16t_ref, wage_ref, bias_ref, gb_ref, out_ref):
    a = ages_ref[...]
    n = jnp.float32(_BATCH)
    mean = jnp.sum(a) / n
    d = a - mean
    var = jnp.sum(d * d) / n
    gamma = gb_ref[0, 0]
    beta = gb_ref[0, 1]
    an = d * lax.rsqrt(var + _EPS) * gamma + beta
    an = jnp.maximum(an, 0.0)
    uf = jnp.maximum(uf_ref[...], 0.0)
    out_ref[...] = (
        jnp.dot(uf, w16t_ref[...], preferred_element_type=jnp.float32)
        + an * wage_ref[...]
        + bias_ref[...]
    )


def kernel(user_ids, ages, emb_table, bn_gamma, bn_beta, W, b):
    out_dim = W.shape[0]
    w_pad = jnp.zeros((_EMB, W.shape[1]), W.dtype).at[:out_dim].set(W)
    w16t = w_pad[:, :_EMB].T
    wage = w_pad[:, _EMB].reshape(1, _EMB)
    bias = jnp.zeros((1, _EMB), b.dtype).at[0, :out_dim].set(b)
    gb = jnp.stack([bn_gamma[0], bn_beta[0]]).reshape(1, 2)

    uf = _sc_gather(user_ids, emb_table)

    out_pad = pl.pallas_call(
        _tc_body,
        out_shape=jax.ShapeDtypeStruct((_BATCH, _EMB), jnp.float32),
    )(uf, ages.reshape(_BATCH, 1), w16t, wage, bias, gb)
    return out_pad[:, :out_dim]
